# baseline (speedup 1.0000x reference)
import functools

import jax
import jax.numpy as jnp
from jax import lax
from jax.experimental import pallas as pl
from jax.experimental.pallas import tpu as pltpu
from jax.experimental.pallas import tpu_sc as plsc

D = 64
DP = 128
B = 4096 * 200
NC = 2
NS = 16
NW = NC * NS
B_PER_W = B // NW
C = 128
K = 4
KC = K * C
G = B_PER_W // KC
L = 16

_mesh = plsc.VectorSubcoreMesh(core_axis_name="c", subcore_axis_name="s")


@functools.partial(
    pl.kernel,
    mesh=_mesh,
    out_type=jax.ShapeDtypeStruct((2 * B, D), jnp.float32),
    scratch_types=[
        pltpu.VMEM((2, KC), jnp.int32),
        pltpu.VMEM((2, KC, D), jnp.float32),
        pltpu.VMEM((K, C), jnp.int32),
        pltpu.VMEM((2, K, C), jnp.int32),
        pltpu.SemaphoreType.DMA,
        pltpu.SemaphoreType.DMA,
        pltpu.SemaphoreType.DMA,
        pltpu.SemaphoreType.DMA,
        pltpu.SemaphoreType.DMA,
        pltpu.SemaphoreType.DMA,
    ],
    compiler_params=pltpu.CompilerParams(use_tc_tiling_on_sc=False),
)
def _gather_kernel(tok_hbm, table_hbm, out_hbm, idx_v, rows_v, opat_v, oidx_v,
                   sem_i0, sem_i1, sem_g0, sem_g1, sem_s0, sem_s1):
    wid = lax.axis_index("s") * NC + lax.axis_index("c")
    base = wid * B_PER_W
    sem_i = (sem_i0, sem_i1)
    sem_g = (sem_g0, sem_g1)
    sem_s = (sem_s0, sem_s1)

    lanes = lax.iota(jnp.int32, L)
    for j in range(K):
        for t in range(C // L):
            opat_v[j, pl.ds(t * L, L)] = 2 * (j * C + t * L) + 2 * lanes

    def fill_oidx(b, off):
        off2 = 2 * off
        for j in range(K):
            for t in range(C // L):
                oidx_v[b, j, pl.ds(t * L, L)] = (
                    opat_v[j, pl.ds(t * L, L)] + off2)

    def fire_gathers(b):
        for j in range(K):
            pltpu.async_copy(
                table_hbm.at[idx_v.at[b, pl.ds(j * C, C)]],
                rows_v.at[b, pl.ds(j * C, C)],
                sem_g[b],
            )

    def wait_idx(b, off):
        pltpu.make_async_copy(tok_hbm.at[pl.ds(off, KC)],
                              idx_v.at[b], sem_i[b]).wait()

    def drain_gathers(b, off):
        pltpu.make_async_copy(rows_v.at[b],
                              out_hbm.at[pl.ds(2 * off, KC)], sem_g[b]).wait()

    def start_store(b):
        for j in range(K):
            pltpu.async_copy(rows_v.at[b, pl.ds(j * C, C)],
                             out_hbm.at[oidx_v.at[b, j]], sem_s[b])

    def wait_store(b, off):
        pltpu.make_async_copy(rows_v.at[b],
                              out_hbm.at[pl.ds(2 * off, KC)], sem_s[b]).wait()

    def prefetch_idx(b, off):
        pltpu.async_copy(tok_hbm.at[pl.ds(off, KC)], idx_v.at[b], sem_i[b])

    prefetch_idx(0, base)
    prefetch_idx(1, base + KC)
    fill_oidx(0, base)
    fill_oidx(1, base + KC)
    wait_idx(0, base)
    fire_gathers(0)
    wait_idx(1, base + KC)
    fire_gathers(1)
    drain_gathers(0, base)
    prefetch_idx(0, base + 2 * KC)
    start_store(0)

    def grp(s, carry):
        g0 = 2 * s
        off0 = base + g0 * KC
        off1 = off0 + KC
        wait_idx(0, off0)
        wait_store(0, off0)
        fill_oidx(0, off0)
        fire_gathers(0)
        drain_gathers(1, off0 - KC)
        prefetch_idx(1, off1)
        start_store(1)
        wait_idx(1, off1)
        wait_store(1, off1)
        fill_oidx(1, off1)
        fire_gathers(1)
        drain_gathers(0, off0)
        prefetch_idx(0, lax.rem(off1 + KC, B))
        start_store(0)
        return carry

    lax.fori_loop(1, G // 2, grp, 0)

    drain_gathers(1, base + (G - 1) * KC)
    start_store(1)
    wait_store(0, base)
    wait_store(1, base)
    wait_idx(0, base)


def kernel(tokens, table):
    half = table.shape[0] // 2
    table_p = jnp.concatenate(
        [jnp.pad(table[:half], ((0, 0), (0, DP - D))),
         jnp.pad(table[half:], ((0, 0), (0, DP - D)))], axis=0)
    table_v = table_p.reshape(2 * table.shape[0], D)
    tok = tokens.reshape(-1).astype(jnp.int32) * 2
    out_p = _gather_kernel(tok, table_v)
    return out_p.reshape(tokens.shape[0], tokens.shape[1], DP)[:, :, :D]


# speedup vs baseline: 1.2744x; 1.2744x over previous
import functools

import jax
import jax.numpy as jnp
from jax import lax
from jax.experimental import pallas as pl
from jax.experimental.pallas import tpu as pltpu
from jax.experimental.pallas import tpu_sc as plsc

D = 64
DP = 128
B = 4096 * 200
NC = 2
NS = 16
NW = NC * NS
B_PER_W = B // NW
C = 128
K = 4
KC = K * C
G = B_PER_W // KC
L = 16

_mesh = plsc.VectorSubcoreMesh(core_axis_name="c", subcore_axis_name="s")


@functools.partial(
    pl.kernel,
    mesh=_mesh,
    out_type=jax.ShapeDtypeStruct((2 * B, D), jnp.float32),
    scratch_types=[
        pltpu.VMEM((2, KC), jnp.int32),
        pltpu.VMEM((2, KC, D), jnp.float32),
        pltpu.VMEM((K, C), jnp.int32),
        pltpu.VMEM((2, K, C), jnp.int32),
        pltpu.SemaphoreType.DMA,
        pltpu.SemaphoreType.DMA,
        pltpu.SemaphoreType.DMA,
        pltpu.SemaphoreType.DMA,
        pltpu.SemaphoreType.DMA,
        pltpu.SemaphoreType.DMA,
    ],
    compiler_params=pltpu.CompilerParams(use_tc_tiling_on_sc=False),
)
def _gather_kernel(tok_hbm, table_hbm, out_hbm, idx_v, rows_v, opat_v, oidx_v,
                   sem_i0, sem_i1, sem_g0, sem_g1, sem_s0, sem_s1):
    wid = lax.axis_index("s") * NC + lax.axis_index("c")
    base = wid * B_PER_W
    sem_i = (sem_i0, sem_i1)
    sem_g = (sem_g0, sem_g1)
    sem_s = (sem_s0, sem_s1)

    lanes = lax.iota(jnp.int32, L)
    for j in range(K):
        for t in range(C // L):
            opat_v[j, pl.ds(t * L, L)] = 2 * (j * C + t * L) + 2 * lanes

    def fill_oidx(b, off):
        off2 = 2 * off
        for j in range(K):
            for t in range(C // L):
                oidx_v[b, j, pl.ds(t * L, L)] = (
                    opat_v[j, pl.ds(t * L, L)] + off2)

    def fire_gathers(b):
        for j in range(K):
            pltpu.async_copy(
                table_hbm.at[idx_v.at[b, pl.ds(j * C, C)]],
                rows_v.at[b, pl.ds(j * C, C)],
                sem_g[b],
            )

    def wait_idx(b, off):
        pltpu.make_async_copy(tok_hbm.at[pl.ds(off, KC)],
                              idx_v.at[b], sem_i[b]).wait()

    def drain_gathers(b, off):
        pltpu.make_async_copy(rows_v.at[b],
                              out_hbm.at[pl.ds(2 * off, KC)], sem_g[b]).wait()

    def start_store(b):
        for j in range(K):
            pltpu.async_copy(rows_v.at[b, pl.ds(j * C, C)],
                             out_hbm.at[oidx_v.at[b, j]], sem_s[b])

    def wait_store(b, off):
        pltpu.make_async_copy(rows_v.at[b],
                              out_hbm.at[pl.ds(2 * off, KC)], sem_s[b]).wait()

    def prefetch_idx(b, off):
        pltpu.async_copy(tok_hbm.at[pl.ds(off, KC)], idx_v.at[b], sem_i[b])

    prefetch_idx(0, base)
    prefetch_idx(1, base + KC)
    fill_oidx(0, base)
    fill_oidx(1, base + KC)
    wait_idx(0, base)
    fire_gathers(0)
    wait_idx(1, base + KC)
    fire_gathers(1)
    drain_gathers(0, base)
    prefetch_idx(0, base + 2 * KC)
    start_store(0)

    def grp(s, carry):
        g0 = 2 * s
        off0 = base + g0 * KC
        off1 = off0 + KC
        wait_idx(0, off0)
        wait_store(0, off0)
        fill_oidx(0, off0)
        fire_gathers(0)
        drain_gathers(1, off0 - KC)
        prefetch_idx(1, off1)
        start_store(1)
        wait_idx(1, off1)
        wait_store(1, off1)
        fill_oidx(1, off1)
        fire_gathers(1)
        drain_gathers(0, off0)
        prefetch_idx(0, lax.rem(off1 + KC, B))
        start_store(0)
        return carry

    lax.fori_loop(1, G // 2, grp, 0)

    drain_gathers(1, base + (G - 1) * KC)
    start_store(1)
    wait_store(0, base)
    wait_store(1, base)
    wait_idx(0, base)


def kernel(tokens, table):
    table_p = jnp.pad(table, ((0, 0), (0, DP - D)))
    table_v = table_p.reshape(2 * table.shape[0], D)
    tok = tokens.reshape(-1).astype(jnp.int32) * 2
    out_p = _gather_kernel(tok, table_v)
    return out_p.reshape(tokens.shape[0], tokens.shape[1], DP)[:, :, :D]
